# baseline (speedup 1.0000x reference)
import jax
import jax.numpy as jnp
from jax import lax
from jax.experimental import pallas as pl
from jax.experimental.pallas import tpu as pltpu

_TP = 2048


def _onehot_mm_kernel(idx_ref, x_ref, pe_ref, o_ref):
    tp = x_ref.shape[0]
    table_len = pe_ref.shape[0]
    one_hot = (idx_ref[...] ==
               lax.broadcasted_iota(jnp.int32, (tp, table_len), 1)
               ).astype(jnp.float32)
    rows = jnp.dot(one_hot, pe_ref[...], preferred_element_type=jnp.float32)
    o_ref[...] = x_ref[...] + rows


@jax.jit
def _pe_gather_add(x2d, idx2d, pe_bf16):
    bp, d = x2d.shape
    table_len = pe_bf16.shape[0]
    nb = bp // _TP
    nj = nb // 2

    cost = pl.CostEstimate(
        flops=2 * bp * table_len * d + bp * d,
        transcendentals=0,
        bytes_accessed=2 * bp * d * 4 + table_len * d * 4 + bp * 4,
    )
    return pl.pallas_call(
        _onehot_mm_kernel,
        grid=(2, nj),
        in_specs=[
            pl.BlockSpec((_TP, 1), lambda c, j: (c * nj + j, 0)),
            pl.BlockSpec((_TP, d), lambda c, j: (c * nj + j, 0)),
            pl.BlockSpec((table_len, d), lambda c, j: (0, 0)),
        ],
        out_specs=pl.BlockSpec((_TP, d), lambda c, j: (c * nj + j, 0)),
        out_shape=jax.ShapeDtypeStruct((bp, d), x2d.dtype),
        compiler_params=pltpu.CompilerParams(
            dimension_semantics=("parallel", "arbitrary"),
            vmem_limit_bytes=48 * 2**20),
        cost_estimate=cost,
    )(idx2d, x2d, pe_bf16)


def kernel(x, pe_param, indices):
    B, P, D = x.shape
    pe_bf16 = pe_param[0]
    x2d = x.reshape(B * P, D)
    idx2d = indices.reshape(B * P, 1).astype(jnp.int32)
    out2d = _pe_gather_add(x2d, idx2d, pe_bf16)
    return out2d.reshape(B, P, D)


# speedup vs baseline: 1.0356x; 1.0356x over previous
import jax
import jax.numpy as jnp
from jax import lax
from jax.experimental import pallas as pl
from jax.experimental.pallas import tpu as pltpu

_TP = 2048


def _outer_kernel(idx_hbm, x_hbm, pe_hbm, o_hbm, pe_raw, pe_bf, sem):
    copy = pltpu.make_async_copy(pe_hbm, pe_raw, sem)
    copy.start()
    copy.wait()
    pe_bf[...] = pe_raw[...].astype(jnp.bfloat16)

    bp, d = x_hbm.shape
    table_len = pe_raw.shape[0]
    nb = bp // _TP

    def _body(idx_ref, x_ref, o_ref):
        one_hot = (idx_ref[...] ==
                   lax.broadcasted_iota(jnp.int32, (_TP, table_len), 1)
                   ).astype(jnp.bfloat16)
        rows = jnp.dot(one_hot, pe_bf[...],
                       preferred_element_type=jnp.float32)
        o_ref[...] = x_ref[...] + rows

    pipe = pltpu.emit_pipeline(
        _body,
        grid=(nb,),
        in_specs=[
            pl.BlockSpec((_TP, 1), lambda i: (i, 0)),
            pl.BlockSpec((_TP, d), lambda i: (i, 0)),
        ],
        out_specs=[pl.BlockSpec((_TP, d), lambda i: (i, 0))],
    )
    pipe(idx_hbm, x_hbm, o_hbm)


@jax.jit
def _pe_gather_add(x2d, idx2d, pe):
    bp, d = x2d.shape
    table_len = pe.shape[0]

    cost = pl.CostEstimate(
        flops=2 * bp * table_len * d + bp * d,
        transcendentals=0,
        bytes_accessed=2 * bp * d * 4 + table_len * d * 4 + bp * 4,
    )
    return pl.pallas_call(
        _outer_kernel,
        in_specs=[
            pl.BlockSpec(memory_space=pl.ANY),
            pl.BlockSpec(memory_space=pl.ANY),
            pl.BlockSpec(memory_space=pl.ANY),
        ],
        out_specs=pl.BlockSpec(memory_space=pl.ANY),
        out_shape=jax.ShapeDtypeStruct((bp, d), x2d.dtype),
        scratch_shapes=[
            pltpu.VMEM((table_len, d), jnp.float32),
            pltpu.VMEM((table_len, d), jnp.bfloat16),
            pltpu.SemaphoreType.DMA,
        ],
        compiler_params=pltpu.CompilerParams(
            vmem_limit_bytes=48 * 2**20),
        cost_estimate=cost,
    )(idx2d, x2d, pe)


def kernel(x, pe_param, indices):
    B, P, D = x.shape
    x2d = x.reshape(B * P, D)
    idx2d = indices.reshape(B * P, 1).astype(jnp.int32)
    out2d = _pe_gather_add(x2d, idx2d, pe_param[0])
    return out2d.reshape(B, P, D)
